# baseline (speedup 1.0000x reference)
import functools

import jax
import jax.numpy as jnp
from jax import lax
from jax.experimental import pallas as pl
from jax.experimental.pallas import tpu as pltpu
from jax.experimental.pallas import tpu_sc as plsc

_B, _N, _C = 4, 2048, 256
_H, _D = 8, 64
_HD = _H * _D
_K = 32
_TR = 256
_PR = 512
_NT = _N // _TR
_NROW = _B * _N
_NW = 32
_RPW = _NROW // _NW


def _sim_kernel(xr_ref, xb_ref, s_ref):
    rt = pl.program_id(1)
    sim = jax.lax.dot_general(xr_ref[0], xb_ref[0], (((1,), (1,)), ((), ())),
                              preferred_element_type=jnp.float32)
    rows = rt * _TR + jax.lax.broadcasted_iota(jnp.int32, (_TR, _N), 0)
    cols = jax.lax.broadcasted_iota(jnp.int32, (_TR, _N), 1)
    s_ref[...] = sim - jnp.where(rows == cols, 1e9, 0.0).astype(jnp.float32)


def _sim(x):
    return pl.pallas_call(
        _sim_kernel,
        grid=(_B, _NT),
        in_specs=[pl.BlockSpec((1, _TR, _C), lambda b, r: (b, r, 0)),
                  pl.BlockSpec((1, _N, _C), lambda b, r: (b, 0, 0))],
        out_specs=pl.BlockSpec((_TR, _N), lambda b, r: (b * _NT + r, 0)),
        out_shape=jax.ShapeDtypeStruct((_NROW, _N), jnp.float32),
    )(x, x)


def _sort16(v):
    s, _ = plsc.sort_key_val(v, v, descending=True)
    return s


def _topk_thresholds(s_flat):
    mesh = plsc.VectorSubcoreMesh(core_axis_name="c", subcore_axis_name="s")

    @functools.partial(
        pl.kernel,
        out_type=jax.ShapeDtypeStruct((_NROW,), jnp.float32),
        mesh=mesh,
        compiler_params=pltpu.CompilerParams(needs_layout_passes=False),
        scratch_types=[
            pltpu.VMEM((_N,), jnp.float32),
            pltpu.VMEM((_N,), jnp.float32),
            pltpu.VMEM((_N + 32,), jnp.float32),
            pltpu.VMEM((_RPW,), jnp.float32),
            pltpu.SemaphoreType.DMA,
            pltpu.SemaphoreType.DMA,
        ],
    )
    def k(s_hbm, t_hbm, rowa, rowb, cand, tbuf, sema, semb):
        wid = lax.axis_index("s") * 2 + lax.axis_index("c")
        base = wid * _RPW
        neg = jnp.full((16,), -jnp.inf, dtype=jnp.float32)
        lanes = lax.iota(jnp.int32, 16)

        def process(row_ref, rl, tvec):
            def p1(i, c):
                m1, m2 = c
                v = row_ref[pl.ds(i * 16, 16)]
                m2 = jnp.maximum(m2, jnp.minimum(m1, v))
                m1 = jnp.maximum(m1, v)
                return m1, m2

            _, m2 = lax.fori_loop(0, _N // 16, p1, (neg, neg))
            t0 = jnp.min(m2)

            def p2(i, off):
                v = row_ref[pl.ds(i * 16, 16)]
                msk = v >= t0
                plsc.store_compressed(cand.at[pl.ds(off, 16)], v, mask=msk)
                return off + jnp.sum(msk.astype(jnp.int32))

            off = lax.fori_loop(0, _N // 16, p2, jnp.int32(0))
            cand[pl.ds(off, 16)] = neg

            def p3(i, ab):
                a, b = ab
                vs = _sort16(cand[pl.ds(i * 16, 16)])
                x1 = jnp.maximum(b, lax.rev(vs, (0,)))
                x1r = lax.rev(_sort16(x1), (0,))
                hi = jnp.maximum(a, x1r)
                lo = jnp.minimum(a, x1r)
                return _sort16(hi), _sort16(lo)

            nv = (off + 15) // 16
            _, b = lax.fori_loop(0, nv, p3, (neg, neg))
            t = jnp.min(b)

            tvec = jnp.where(lanes == rl % 16, t, tvec)

            @pl.when(rl % 16 == 15)
            def _():
                tbuf[pl.ds(rl - 15, 16)] = tvec

            return tvec

        pltpu.async_copy(s_hbm.at[base], rowa, sema)

        def row_pair(j, tvec):
            r0 = base + 2 * j
            pltpu.async_copy(s_hbm.at[r0 + 1], rowb, semb)
            pltpu.make_async_copy(s_hbm.at[r0], rowa, sema).wait()
            tvec = process(rowa, 2 * j, tvec)

            @pl.when(j < _RPW // 2 - 1)
            def _():
                pltpu.async_copy(s_hbm.at[r0 + 2], rowa, sema)

            pltpu.make_async_copy(s_hbm.at[r0 + 1], rowb, semb).wait()
            tvec = process(rowb, 2 * j + 1, tvec)
            return tvec

        lax.fori_loop(0, _RPW // 2, row_pair, jnp.zeros((16,), jnp.float32))
        pltpu.sync_copy(tbuf, t_hbm.at[pl.ds(base, _RPW)])

    return k(s_flat).reshape(_NROW, 1)


def _proj_kernel(x_ref, wq_ref, wk_ref, wv_ref, ws_ref,
                 q_ref, k_ref, v_ref, s_ref):
    x = x_ref[...]
    q_ref[...] = jnp.dot(x, wq_ref[...], preferred_element_type=jnp.float32)
    k_ref[...] = jnp.dot(x, wk_ref[...], preferred_element_type=jnp.float32)
    v_ref[...] = jnp.dot(x, wv_ref[...], preferred_element_type=jnp.float32)
    s_ref[...] = jnp.dot(x, ws_ref[...], preferred_element_type=jnp.float32)


def _project(xf, Wq, Wk, Wv, Wskip):
    wspec = pl.BlockSpec((_C, _HD), lambda i: (0, 0))
    rspec = pl.BlockSpec((_PR, _HD), lambda i: (i, 0))
    return pl.pallas_call(
        _proj_kernel,
        grid=(_NROW // _PR,),
        in_specs=[pl.BlockSpec((_PR, _C), lambda i: (i, 0)),
                  wspec, wspec, wspec, wspec],
        out_specs=[rspec, rspec, rspec, rspec],
        out_shape=[jax.ShapeDtypeStruct((_NROW, _HD), jnp.float32)] * 4,
    )(xf, Wq, Wk, Wv, Wskip)


def _attn_kernel(s_ref, t_ref, q_ref, k_ref, v_ref, skip_ref, o_ref):
    maskf = (s_ref[...] >= t_ref[...]).astype(jnp.float32)
    skip = skip_ref[0]
    for h in range(_H):
        sl = slice(h * _D, (h + 1) * _D)
        qh = q_ref[0][:, sl]
        kh = k_ref[0][:, sl]
        vh = v_ref[0][:, sl]
        logits = jax.lax.dot_general(qh, kh, (((1,), (1,)), ((), ())),
                                     preferred_element_type=jnp.float32)
        m = jnp.max(logits, axis=1, keepdims=True)
        e = jnp.exp(logits - m) * maskf
        ssum = jnp.sum(e, axis=1, keepdims=True)
        oh = jnp.dot(e, vh, preferred_element_type=jnp.float32)
        o_ref[0, :, sl] = oh * (1.0 / (ssum + 1e-16)) + skip[:, sl]


def _attention(s_flat, thr, q, k, v, skip):
    row3 = pl.BlockSpec((1, _TR, _HD), lambda b, r: (b, r, 0))
    full3 = pl.BlockSpec((1, _N, _HD), lambda b, r: (b, 0, 0))
    return pl.pallas_call(
        _attn_kernel,
        grid=(_B, _NT),
        in_specs=[pl.BlockSpec((_TR, _N), lambda b, r: (b * _NT + r, 0)),
                  pl.BlockSpec((_TR, 1), lambda b, r: (b * _NT + r, 0)),
                  row3, full3, full3, row3],
        out_specs=row3,
        out_shape=jax.ShapeDtypeStruct((_B, _N, _HD), jnp.float32),
    )(s_flat, thr, q, k, v, skip)


def kernel(x, Wq, Wk, Wv, Wskip):
    xf = x.reshape(_NROW, _C)
    s_flat = _sim(x)
    thr = _topk_thresholds(s_flat)
    q, k, v, skip = _project(xf, Wq * jnp.float32(1.0 / (_D ** 0.5)),
                             Wk, Wv, Wskip)
    q = q.reshape(_B, _N, _HD)
    k = k.reshape(_B, _N, _HD)
    v = v.reshape(_B, _N, _HD)
    skip = skip.reshape(_B, _N, _HD)
    return _attention(s_flat, thr, q, k, v, skip)


# speedup vs baseline: 1.3326x; 1.3326x over previous
import functools

import jax
import jax.numpy as jnp
from jax import lax
from jax.experimental import pallas as pl
from jax.experimental.pallas import tpu as pltpu
from jax.experimental.pallas import tpu_sc as plsc

_B, _N, _C = 4, 2048, 256
_H, _D = 8, 64
_HD = _H * _D
_K = 32
_TR = 256
_PR = 512
_NT = _N // _TR
_NROW = _B * _N
_NW = 32
_RPW = _NROW // _NW
_NCH = _N // 16


def _sim_kernel(xr_ref, xb_ref, s_ref, p_ref):
    rt = pl.program_id(1)
    sim = jax.lax.dot_general(xr_ref[0], xb_ref[0], (((1,), (1,)), ((), ())),
                              preferred_element_type=jnp.float32)
    rows = rt * _TR + jax.lax.broadcasted_iota(jnp.int32, (_TR, _N), 0)
    cols = jax.lax.broadcasted_iota(jnp.int32, (_TR, _N), 1)
    sim = sim - jnp.where(rows == cols, 1e9, 0.0).astype(jnp.float32)
    s_ref[...] = sim
    p_ref[0] = jnp.max(sim.reshape(_TR // 16, 16, _N), axis=1)


def _sim(x):
    return pl.pallas_call(
        _sim_kernel,
        grid=(_B, _NT),
        in_specs=[pl.BlockSpec((1, _TR, _C), lambda b, r: (b, r, 0)),
                  pl.BlockSpec((1, _N, _C), lambda b, r: (b, 0, 0))],
        out_specs=[pl.BlockSpec((_TR, _N), lambda b, r: (b * _NT + r, 0)),
                   pl.BlockSpec((1, _TR // 16, _N), lambda b, r: (b, r, 0))],
        out_shape=[jax.ShapeDtypeStruct((_NROW, _N), jnp.float32),
                   jax.ShapeDtypeStruct((_B, _NCH, _N), jnp.float32)],
    )(x, x)


def _sort16(v):
    s, _ = plsc.sort_key_val(v, v, descending=True)
    return s


def _topk_thresholds(s_flat, p):
    mesh = plsc.VectorSubcoreMesh(core_axis_name="c", subcore_axis_name="s")

    @functools.partial(
        pl.kernel,
        out_type=jax.ShapeDtypeStruct((_NROW,), jnp.float32),
        mesh=mesh,
        compiler_params=pltpu.CompilerParams(needs_layout_passes=False),
        scratch_types=[
            pltpu.VMEM((_N,), jnp.float32),
            pltpu.VMEM((_N,), jnp.float32),
            pltpu.VMEM((_NCH, _RPW), jnp.float32),
            pltpu.VMEM((_N + 16,), jnp.float32),
            pltpu.VMEM((_NCH + 16,), jnp.int32),
            pltpu.VMEM((_RPW,), jnp.float32),
            pltpu.SemaphoreType.DMA,
            pltpu.SemaphoreType.DMA,
        ],
    )
    def k(s_hbm, p_hbm, t_hbm, rowa, rowb, mslab, cand, cidx, tbuf,
          sema, semb):
        wid = lax.axis_index("s") * 2 + lax.axis_index("c")
        base = wid * _RPW
        bi = wid // (_N // _RPW)
        c0 = (wid % (_N // _RPW)) * _RPW
        neg = jnp.full((16,), -jnp.inf, dtype=jnp.float32)
        zeros_i = jnp.zeros((16,), jnp.int32)
        lanes = lax.iota(jnp.int32, 16)

        pltpu.sync_copy(p_hbm.at[bi, :, pl.ds(c0, _RPW)], mslab)

        def process(row_ref, rl, tvec):
            rl_vec = jnp.full((16,), rl, jnp.int32)

            m1a, m2a = neg, neg
            m1b, m2b = neg, neg
            for u in range(8):
                v = plsc.load_gather(mslab, [lanes + u * 16, rl_vec])
                if u % 2 == 0:
                    m2a = jnp.maximum(m2a, jnp.minimum(m1a, v))
                    m1a = jnp.maximum(m1a, v)
                else:
                    m2b = jnp.maximum(m2b, jnp.minimum(m1b, v))
                    m1b = jnp.maximum(m1b, v)
            m1 = jnp.maximum(m1a, m1b)
            m2 = jnp.maximum(jnp.minimum(m1a, m1b),
                             jnp.maximum(m2a, m2b))
            t0 = jnp.min(m2)

            offc = jnp.int32(0)
            for u in range(8):
                mv = plsc.load_gather(mslab, [lanes + u * 16, rl_vec])
                cm = mv >= t0
                plsc.store_compressed(cidx.at[pl.ds(offc, 16)],
                                      lanes + u * 16, mask=cm)
                offc = offc + plsc.all_reduce_population_count(cm)[0]
            cidx[pl.ds(offc, 16)] = zeros_i

            def p2(jj, off):
                cv = cidx[pl.ds(jj * 16, 16)]
                for l in range(16):
                    g = cv[l]
                    v = row_ref[pl.ds(g * 16, 16)]
                    msk = jnp.logical_and(v >= t0, jj * 16 + l < offc)
                    plsc.store_compressed(cand.at[pl.ds(off, 16)], v,
                                          mask=msk)
                    off = off + plsc.all_reduce_population_count(msk)[0]
                return off

            off = lax.fori_loop(0, (offc + 15) // 16, p2, jnp.int32(0))
            cand[pl.ds(off, 16)] = neg

            def p3(i, ab):
                a, b = ab
                vs = _sort16(cand[pl.ds(i * 16, 16)])
                x1 = jnp.maximum(b, lax.rev(vs, (0,)))
                x1r = lax.rev(_sort16(x1), (0,))
                hi = jnp.maximum(a, x1r)
                lo = jnp.minimum(a, x1r)
                return _sort16(hi), _sort16(lo)

            _, b = lax.fori_loop(0, (off + 15) // 16, p3, (neg, neg))
            t = jnp.min(b)

            tvec = jnp.where(lanes == rl % 16, t, tvec)

            @pl.when(rl % 16 == 15)
            def _():
                tbuf[pl.ds(rl - 15, 16)] = tvec

            return tvec

        pltpu.async_copy(s_hbm.at[base], rowa, sema)

        def row_pair(j, tvec):
            r0 = base + 2 * j
            pltpu.async_copy(s_hbm.at[r0 + 1], rowb, semb)
            pltpu.make_async_copy(s_hbm.at[r0], rowa, sema).wait()
            tvec = process(rowa, 2 * j, tvec)

            @pl.when(j < _RPW // 2 - 1)
            def _():
                pltpu.async_copy(s_hbm.at[r0 + 2], rowa, sema)

            pltpu.make_async_copy(s_hbm.at[r0 + 1], rowb, semb).wait()
            tvec = process(rowb, 2 * j + 1, tvec)
            return tvec

        lax.fori_loop(0, _RPW // 2, row_pair, jnp.zeros((16,), jnp.float32))
        pltpu.sync_copy(tbuf, t_hbm.at[pl.ds(base, _RPW)])

    return k(s_flat, p).reshape(_NROW, 1)


def _proj_kernel(x_ref, wq_ref, wk_ref, wv_ref, ws_ref,
                 q_ref, k_ref, v_ref, s_ref):
    x = x_ref[...]
    q_ref[...] = jnp.dot(x, wq_ref[...], preferred_element_type=jnp.float32)
    k_ref[...] = jnp.dot(x, wk_ref[...], preferred_element_type=jnp.float32)
    v_ref[...] = jnp.dot(x, wv_ref[...], preferred_element_type=jnp.float32)
    s_ref[...] = jnp.dot(x, ws_ref[...], preferred_element_type=jnp.float32)


def _project(xf, Wq, Wk, Wv, Wskip):
    wspec = pl.BlockSpec((_C, _HD), lambda i: (0, 0))
    rspec = pl.BlockSpec((_PR, _HD), lambda i: (i, 0))
    return pl.pallas_call(
        _proj_kernel,
        grid=(_NROW // _PR,),
        in_specs=[pl.BlockSpec((_PR, _C), lambda i: (i, 0)),
                  wspec, wspec, wspec, wspec],
        out_specs=[rspec, rspec, rspec, rspec],
        out_shape=[jax.ShapeDtypeStruct((_NROW, _HD), jnp.float32)] * 4,
    )(xf, Wq, Wk, Wv, Wskip)


def _attn_kernel(s_ref, t_ref, q_ref, k_ref, v_ref, skip_ref, o_ref):
    maskf = (s_ref[...] >= t_ref[...]).astype(jnp.float32)
    skip = skip_ref[0]
    for h in range(_H):
        sl = slice(h * _D, (h + 1) * _D)
        qh = q_ref[0][:, sl]
        kh = k_ref[0][:, sl]
        vh = v_ref[0][:, sl]
        logits = jax.lax.dot_general(qh, kh, (((1,), (1,)), ((), ())),
                                     preferred_element_type=jnp.float32)
        m = jnp.max(logits, axis=1, keepdims=True)
        e = jnp.exp(logits - m) * maskf
        ssum = jnp.sum(e, axis=1, keepdims=True)
        oh = jnp.dot(e, vh, preferred_element_type=jnp.float32)
        o_ref[0, :, sl] = oh * (1.0 / (ssum + 1e-16)) + skip[:, sl]


def _attention(s_flat, thr, q, k, v, skip):
    row3 = pl.BlockSpec((1, _TR, _HD), lambda b, r: (b, r, 0))
    full3 = pl.BlockSpec((1, _N, _HD), lambda b, r: (b, 0, 0))
    return pl.pallas_call(
        _attn_kernel,
        grid=(_B, _NT),
        in_specs=[pl.BlockSpec((_TR, _N), lambda b, r: (b * _NT + r, 0)),
                  pl.BlockSpec((_TR, 1), lambda b, r: (b * _NT + r, 0)),
                  row3, full3, full3, row3],
        out_specs=row3,
        out_shape=jax.ShapeDtypeStruct((_B, _N, _HD), jnp.float32),
    )(s_flat, thr, q, k, v, skip)


def kernel(x, Wq, Wk, Wv, Wskip):
    xf = x.reshape(_NROW, _C)
    s_flat, p = _sim(x)
    thr = _topk_thresholds(s_flat, p)
    q, k, v, skip = _project(xf, Wq * jnp.float32(1.0 / (_D ** 0.5)),
                             Wk, Wv, Wskip)
    q = q.reshape(_B, _N, _HD)
    k = k.reshape(_B, _N, _HD)
    v = v.reshape(_B, _N, _HD)
    skip = skip.reshape(_B, _N, _HD)
    return _attention(s_flat, thr, q, k, v, skip)
